# baseline (speedup 1.0000x reference)
import functools

import jax
import jax.numpy as jnp
from jax import lax
from jax.experimental import pallas as pl
from jax.experimental.pallas import tpu as pltpu

D = 768
HID = 256
TOTAL = 8
MLP_HID = 1536
S = 2048
T = 256
NB = S // T
BT = 256
NJ = S // BT
NBLK = TOTAL * NJ
GROWS = (NBLK + TOTAL) * BT


def _gelu(x):
    return 0.5 * x * (1.0 + jax.lax.erf(x * 0.7071067811865476))


def _ln(x, g, b, eps=1e-5):
    m = jnp.mean(x, axis=-1, keepdims=True)
    v = jnp.mean((x - m) ** 2, axis=-1, keepdims=True)
    return (x - m) / jnp.sqrt(v + eps) * g + b


def _bf(x):
    return x.astype(jnp.bfloat16)


def _dot(a, b):
    return jnp.dot(a, b, preferred_element_type=jnp.float32)


def _k1_body(x_ref, rW1, rb1, rW2, rb2, rW3, rb3, temp,
             Wpre_h, bpre, gpre, bepre,
             A_ref, w_ref, pos_ref, src_ref, meta_ref, loss_ref,
             freq_acc, run_cnt, Wpre_b, stg0, stg1, sem):
    i = pl.program_id(0)

    @pl.when(i == 0)
    def _():
        c0 = pltpu.make_async_copy(Wpre_h.at[0], stg0, sem.at[0])
        c1 = pltpu.make_async_copy(Wpre_h.at[1], stg1, sem.at[1])
        c0.start()
        c1.start()
        c0.wait()
        Wpre_b[:, :D] = _bf(stg0[...])
        c1.wait()
        Wpre_b[:, D:] = _bf(stg1[...])

    xb = x_ref[...]

    h = _gelu(_dot(xb, rW1[...]) + rb1[...])
    h = _gelu(_dot(h, rW2[...]) + rb2[...])
    s = _dot(h, rW3[...]) + rb3[...]

    s_max = jnp.max(s, axis=-1, keepdims=True)
    e = jnp.exp(s - s_max)
    p_lb = e / jnp.sum(e, axis=-1, keepdims=True)
    part = jnp.sum(p_lb, axis=0, keepdims=True)

    @pl.when(i == 0)
    def _():
        freq_acc[...] = part

    @pl.when(i > 0)
    def _():
        freq_acc[...] = freq_acc[...] + part

    @pl.when(i == NB - 1)
    def _():
        freq = freq_acc[...] / float(S)
        mu = jnp.mean(freq)
        var = jnp.sum((freq - mu) ** 2) / (TOTAL - 1)
        loss_ref[...] = jnp.reshape(TOTAL * var, (1, 1))

    st = s / temp[...]
    st_max = jnp.max(st, axis=-1, keepdims=True)
    et = jnp.exp(st - st_max)
    pt = et / jnp.sum(et, axis=-1, keepdims=True)

    idx = lax.broadcasted_iota(jnp.int32, (T, TOTAL), 1)
    m1 = jnp.max(pt, axis=-1, keepdims=True)
    i1 = jnp.min(jnp.where(pt == m1, idx, TOTAL), axis=-1, keepdims=True)
    pt2 = jnp.where(idx == i1, -jnp.inf, pt)
    m2 = jnp.max(pt2, axis=-1, keepdims=True)
    i2 = jnp.min(jnp.where(pt2 == m2, idx, TOTAL), axis=-1, keepdims=True)

    M = ((idx == i1) | (idx == i2)).astype(jnp.float32)
    r_io = lax.broadcasted_iota(jnp.int32, (T, T), 0)
    c_io = lax.broadcasted_iota(jnp.int32, (T, T), 1)
    ltri = (r_io > c_io).astype(jnp.bfloat16)
    excl = _dot(ltri, _bf(M))
    rc = jnp.where(i == 0, jnp.zeros((1, TOTAL), jnp.float32), run_cnt[...])
    rank = excl + rc
    cnt = rc + jnp.sum(M, axis=0, keepdims=True)
    run_cnt[...] = cnt

    rank1 = jnp.sum(rank * (idx == i1).astype(jnp.float32), -1, keepdims=True)
    rank2 = jnp.sum(rank * (idx == i2).astype(jnp.float32), -1, keepdims=True)
    i1f = i1.astype(jnp.float32)
    i2f = i2.astype(jnp.float32)
    tok = (lax.broadcasted_iota(jnp.int32, (T, 1), 0) + i * T).astype(jnp.float32)
    pos1 = i1f * S + rank1
    pos2 = i2f * S + rank2
    src1 = jnp.floor(i1f / 4.0) * S + tok
    src2 = jnp.floor(i2f / 4.0) * S + tok
    z = jnp.zeros((T, 1), jnp.float32)
    V = jnp.concatenate([pos1, pos2, src1, src2, m1, m2, z, z], axis=1)
    Vt = jnp.transpose(V)
    pos_ref[...] = Vt[0:2].astype(jnp.int32)
    src_ref[...] = Vt[2:4].astype(jnp.int32)
    w_ref[...] = Vt[4:6]

    @pl.when(i == NB - 1)
    def _():
        ncol = jnp.transpose(cnt)
        jj = lax.broadcasted_iota(jnp.int32, (TOTAL, NJ), 1)
        active = (jj.astype(jnp.float32) * BT < ncol)
        pp = lax.broadcasted_iota(jnp.int32, (TOTAL, NJ), 0)
        bidx = jnp.where(active, pp * NJ + jj, NBLK + pp)
        meta_ref[...] = jnp.stack(
            [bidx, active.astype(jnp.int32)], axis=0)

    zpre = _dot(_bf(xb), Wpre_b[...])
    for a in range(2):
        za = zpre[:, a * D:(a + 1) * D] + bpre[a]
        za = _ln(za, gpre[a], bepre[a])
        za = _gelu(za) if a == 0 else jnp.maximum(za, 0.0)
        A_ref[a] = za


def _k1(xf, rW1, rb1, rW2, rb2, rW3, rb3, temp, W_pre, b_pre, g_pre, be_pre):
    full = lambda shape: pl.BlockSpec(shape, lambda i: (0,) * len(shape))
    return pl.pallas_call(
        _k1_body,
        grid=(NB,),
        in_specs=[
            pl.BlockSpec((T, D), lambda i: (i, 0)),
            full((D, HID)), full((HID,)), full((HID, HID // 2)), full((HID // 2,)),
            full((HID // 2, TOTAL)), full((TOTAL,)), full((1, 1)),
            pl.BlockSpec(memory_space=pl.ANY),
            full((2, D)), full((2, D)), full((2, D)),
        ],
        out_specs=[
            pl.BlockSpec((2, T, D), lambda i: (0, i, 0)),
            pl.BlockSpec((2, T), lambda i: (0, i)),
            pl.BlockSpec((2, T), lambda i: (0, i)),
            pl.BlockSpec((2, T), lambda i: (0, i)),
            pl.BlockSpec((2, TOTAL, NJ), lambda i: (0, 0, 0)),
            pl.BlockSpec((1, 1), lambda i: (0, 0)),
        ],
        out_shape=[
            jax.ShapeDtypeStruct((2, S, D), jnp.float32),
            jax.ShapeDtypeStruct((2, S), jnp.float32),
            jax.ShapeDtypeStruct((2, S), jnp.int32),
            jax.ShapeDtypeStruct((2, S), jnp.int32),
            jax.ShapeDtypeStruct((2, TOTAL, NJ), jnp.int32),
            jax.ShapeDtypeStruct((1, 1), jnp.float32),
        ],
        scratch_shapes=[
            pltpu.VMEM((1, TOTAL), jnp.float32),
            pltpu.VMEM((1, TOTAL), jnp.float32),
            pltpu.VMEM((D, 2 * D), jnp.bfloat16),
            pltpu.VMEM((D, D), jnp.float32),
            pltpu.VMEM((D, D), jnp.float32),
            pltpu.SemaphoreType.DMA((2,)),
        ],
        compiler_params=pltpu.CompilerParams(
            dimension_semantics=("arbitrary",),
        ),
    )(xf, rW1, rb1, rW2, rb2, rW3, rb3, temp.reshape(1, 1),
      W_pre, b_pre, g_pre, be_pre)


def _k3_body(meta_s, G_ref, Wm1_h, bm1, Wm2_h, bm2, Wpo_h, bpo, gpo, bepo,
             Y_ref, Wm1_b, Wm2_b, Wpo_b, stg0, stg1, sem):
    p = pl.program_id(0)
    j = pl.program_id(1)

    @pl.when((p == 0) & (j == 0))
    def _():
        stg = (stg0, stg1)
        chunks = [
            (Wm1_h.at[0], D, MLP_HID, Wm1_b, 0),
            (Wm1_h.at[1], D, MLP_HID, Wm1_b, D),
            (Wm2_h.at[0], MLP_HID, D, Wm2_b, 0),
            (Wm2_h.at[1], MLP_HID, D, Wm2_b, MLP_HID),
            (Wpo_h.at[0], D, D, Wpo_b, 0),
            (Wpo_h.at[1], D, D, Wpo_b, D),
        ]
        copies = []
        for k, (src, r, c, dst, off) in enumerate(chunks):
            copies.append(pltpu.make_async_copy(
                src, stg[k % 2].at[:r, :c], sem.at[k % 2]))
        copies[0].start()
        for k, (src, r, c, dst, off) in enumerate(chunks):
            if k + 1 < len(chunks):
                copies[k + 1].start()
            copies[k].wait()
            dst[off:off + r, :] = _bf(stg[k % 2][:r, :c])

    active = meta_s[NBLK + p * NJ + j]

    @pl.when(active == 1)
    def _():
        m_i = (p // 2) % 2
        o_i = p % 2
        g = _bf(G_ref[...])
        h1 = _dot(g, Wm1_b[pl.ds(m_i * D, D), :])
        h1 = h1 + jnp.where(m_i == 0, bm1[0:1, :], bm1[1:2, :])
        hsel = jnp.where(m_i == 0, _gelu(h1), jnp.maximum(h1, 0.0))
        xm = _dot(_bf(hsel), Wm2_b[pl.ds(m_i * MLP_HID, MLP_HID), :])
        xm = xm + jnp.where(m_i == 0, bm2[0:1, :], bm2[1:2, :])
        zo = _dot(_bf(xm), Wpo_b[pl.ds(o_i * D, D), :])
        zo = zo + jnp.where(o_i == 0, bpo[0:1, :], bpo[1:2, :])
        Y_ref[...] = jnp.where(o_i == 0, _ln(zo, gpo[...], bepo[...]), zo)


def _k3(meta_flat, G, W_m1, b_m1, W_m2, b_m2, W_po, b_po, g_po, be_po):
    full = lambda shape: pl.BlockSpec(
        shape, lambda p, j, m: (0,) * len(shape))
    grid_spec = pltpu.PrefetchScalarGridSpec(
        num_scalar_prefetch=1,
        grid=(TOTAL, NJ),
        in_specs=[
            pl.BlockSpec((BT, D), lambda p, j, m: (m[p * NJ + j], 0)),
            pl.BlockSpec(memory_space=pl.ANY),
            full((2, MLP_HID)),
            pl.BlockSpec(memory_space=pl.ANY),
            full((2, D)),
            pl.BlockSpec(memory_space=pl.ANY),
            full((2, D)),
            full((D,)), full((D,)),
        ],
        out_specs=[
            pl.BlockSpec((BT, D), lambda p, j, m: (m[p * NJ + j], 0)),
        ],
        scratch_shapes=[
            pltpu.VMEM((2 * D, MLP_HID), jnp.bfloat16),
            pltpu.VMEM((2 * MLP_HID, D), jnp.bfloat16),
            pltpu.VMEM((2 * D, D), jnp.bfloat16),
            pltpu.VMEM((MLP_HID, MLP_HID), jnp.float32),
            pltpu.VMEM((MLP_HID, MLP_HID), jnp.float32),
            pltpu.SemaphoreType.DMA((2,)),
        ],
    )
    return pl.pallas_call(
        _k3_body,
        grid_spec=grid_spec,
        out_shape=[jax.ShapeDtypeStruct((GROWS, D), jnp.float32)],
        compiler_params=pltpu.CompilerParams(
            dimension_semantics=("arbitrary", "arbitrary"),
        ),
    )(meta_flat, G, W_m1, b_m1, W_m2, b_m2, W_po, b_po, g_po, be_po)[0]


@functools.partial(jax.jit, static_argnames=())
def kernel(x, rW1, rb1, rW2, rb2, rW3, rb3, temp, W_pre, b_pre, g_pre, be_pre,
           W_m1, b_m1, W_m2, b_m2, W_po, b_po, g_po, be_po):
    xf = x.reshape(S, D)
    A, w2, pos, src, meta, loss = _k1(
        xf, rW1, rb1, rW2, rb2, rW3, rb3, temp, W_pre, b_pre, g_pre, be_pre)
    A4 = A.reshape(2 * S, D)
    pos_f = pos.reshape(2 * S)
    src_f = src.reshape(2 * S)
    w_f = w2.reshape(2 * S)
    meta_flat = meta.reshape(2 * TOTAL * NJ)

    G = jnp.zeros((GROWS, D), jnp.float32).at[pos_f].set(A4[src_f])

    Y = _k3(meta_flat, G, W_m1, b_m1, W_m2, b_m2, W_po, b_po, g_po, be_po)

    out = (w_f[:S, None] * Y[pos_f[:S]] + w_f[S:, None] * Y[pos_f[S:]])

    return out.reshape(1, S, D), loss.reshape(())


# speedup vs baseline: 2.2235x; 2.2235x over previous
import functools

import jax
import jax.numpy as jnp
from jax.experimental import pallas as pl
from jax.experimental.pallas import tpu as pltpu

D = 768
HID = 256
TOTAL = 8
MLP_HID = 1536
S = 2048
T = 256
NB = S // T


def _gelu(x):
    return 0.5 * x * (1.0 + jax.lax.erf(x * 0.7071067811865476))


def _ln(x, g, b, eps=1e-5):
    m = jnp.mean(x, axis=-1, keepdims=True)
    v = jnp.mean((x - m) ** 2, axis=-1, keepdims=True)
    return (x - m) / jnp.sqrt(v + eps) * g + b


def _bf(x):
    return x.astype(jnp.bfloat16)


def _dot(a, b):
    return jnp.dot(a, b, preferred_element_type=jnp.float32)


def _body(x_ref, rW1, rb1, rW2, rb2, rW3, rb3, temp,
          Wpre_h, bpre, gpre, bepre, Wm1_h, bm1, Wm2_h, bm2, Wpo_h, bpo,
          gpo, bepo,
          out_ref, loss_ref,
          freq_acc, A_scr, w_scr, Wpre_b, Wm1_b, Wm2_b, Wpo_b,
          stg0, stg1, sem):
    i = pl.program_id(0)
    stg = (stg0, stg1)

    chunks = [
        (Wm1_h.at[0], D, MLP_HID), (Wm1_h.at[1], D, MLP_HID),
        (Wm2_h.at[0, pl.ds(0, D), :], D, D),
        (Wm2_h.at[0, pl.ds(D, D), :], D, D),
        (Wm2_h.at[1, pl.ds(0, D), :], D, D),
        (Wm2_h.at[1, pl.ds(D, D), :], D, D),
        (Wpo_h.at[0], D, D), (Wpo_h.at[1], D, D),
    ]

    def store_chunk(k, v):
        if k == 0:
            Wm1_b[:, :MLP_HID] = v
        elif k == 1:
            Wm1_b[:, MLP_HID:] = v
        elif k in (2, 3, 4, 5):
            Wm2_b[(k - 2) * D:(k - 1) * D] = v
        elif k == 6:
            Wpo_b[:D] = v
        else:
            Wpo_b[D:] = v

    @pl.when(i == 0)
    def _():
        c0 = pltpu.make_async_copy(Wpre_h.at[0], stg0.at[:D, :D], sem.at[0])
        c1 = pltpu.make_async_copy(Wpre_h.at[1], stg1.at[:D, :D], sem.at[1])
        c0.start()
        c1.start()
        c0.wait()
        Wpre_b[:, :D] = _bf(stg0[:D, :D])
        c1.wait()
        Wpre_b[:, D:] = _bf(stg1[:D, :D])

    for k, (src, r, c) in enumerate(chunks):
        @pl.when(i == k)
        def _(src=src, r=r, c=c, k=k):
            pltpu.make_async_copy(src, stg[k % 2].at[:r, :c],
                                  sem.at[k % 2]).start()

        @pl.when(i == k + 1)
        def _(src=src, r=r, c=c, k=k):
            pltpu.make_async_copy(src, stg[k % 2].at[:r, :c],
                                  sem.at[k % 2]).wait()
            store_chunk(k, _bf(stg[k % 2][:r, :c]))

    @pl.when(i < NB)
    def _():
        xb = x_ref[...]

        h = _gelu(_dot(xb, rW1[...]) + rb1[...])
        h = _gelu(_dot(h, rW2[...]) + rb2[...])
        s = _dot(h, rW3[...]) + rb3[...]

        s_max = jnp.max(s, axis=-1, keepdims=True)
        e = jnp.exp(s - s_max)
        p_lb = e / jnp.sum(e, axis=-1, keepdims=True)
        part = jnp.sum(p_lb, axis=0, keepdims=True)

        @pl.when(i == 0)
        def _():
            freq_acc[...] = part

        @pl.when(i > 0)
        def _():
            freq_acc[...] = freq_acc[...] + part

        @pl.when(i == NB - 1)
        def _():
            freq = freq_acc[...] / float(S)
            mu = jnp.mean(freq)
            var = jnp.sum((freq - mu) ** 2) / (TOTAL - 1)
            loss_ref[...] = jnp.reshape(TOTAL * var, (1, 1))

        st = s / temp[...]
        st_max = jnp.max(st, axis=-1, keepdims=True)
        et = jnp.exp(st - st_max)
        pt = et / jnp.sum(et, axis=-1, keepdims=True)

        idx = jax.lax.broadcasted_iota(jnp.int32, (T, TOTAL), 1)
        m1 = jnp.max(pt, axis=-1, keepdims=True)
        i1 = jnp.min(jnp.where(pt == m1, idx, TOTAL), axis=-1, keepdims=True)
        pt2 = jnp.where(idx == i1, -jnp.inf, pt)
        m2 = jnp.max(pt2, axis=-1, keepdims=True)
        i2 = jnp.min(jnp.where(pt2 == m2, idx, TOTAL), axis=-1, keepdims=True)
        w_scr[pl.ds(i * T, T), :] = pt * ((idx == i1) | (idx == i2)).astype(
            jnp.float32)

        zpre = _dot(_bf(xb), Wpre_b[...])
        for a in range(2):
            z = zpre[:, a * D:(a + 1) * D] + bpre[a]
            z = _ln(z, gpre[a], bepre[a])
            z = _gelu(z) if a == 0 else jnp.maximum(z, 0.0)
            A_scr[pl.ds(a * S + i * T, T), :] = _bf(z)

    @pl.when(i >= NB)
    def _():
        b = i - NB
        w = w_scr[pl.ds(b * T, T), :]
        A0 = A_scr[pl.ds(b * T, T), :]
        A1 = A_scr[pl.ds(S + b * T, T), :]
        Acat = jnp.concatenate([A0, A1], axis=0)
        hcat = _dot(Acat, Wm1_b[...])
        hm0 = _bf(_gelu(hcat[:, :MLP_HID] + bm1[0]))
        hm1 = _bf(jnp.maximum(hcat[:, MLP_HID:] + bm1[1], 0.0))
        xm0 = _dot(hm0, Wm2_b[:MLP_HID]) + bm2[0]
        xm1 = _dot(hm1, Wm2_b[MLP_HID:]) + bm2[1]
        X = jnp.concatenate([xm0, xm1], axis=0)
        z0 = _ln(_dot(_bf(X), Wpo_b[:D]) + bpo[0], gpo[...], bepo[...])
        out0 = jnp.zeros((T, D), jnp.float32)
        u1 = jnp.zeros((T, D), jnp.float32)
        sw1 = jnp.zeros((T, 1), jnp.float32)
        for j in range(4):
            m_i, a_i = j // 2, j % 2
            p0 = a_i * 4 + m_i * 2
            w0 = w[:, p0:p0 + 1]
            w1 = w[:, p0 + 1:p0 + 2]
            out0 = out0 + w0 * z0[j * T:(j + 1) * T]
            u1 = u1 + w1 * X[j * T:(j + 1) * T]
            sw1 = sw1 + w1
        out_ref[...] = out0 + _dot(_bf(u1), Wpo_b[D:]) + sw1 * bpo[1]


@functools.partial(jax.jit, static_argnames=())
def kernel(x, rW1, rb1, rW2, rb2, rW3, rb3, temp, W_pre, b_pre, g_pre, be_pre,
           W_m1, b_m1, W_m2, b_m2, W_po, b_po, g_po, be_po):
    xf = x.reshape(S, D)
    full = lambda shape: pl.BlockSpec(shape, lambda i: (0,) * len(shape))
    hbm = pl.BlockSpec(memory_space=pl.ANY)
    out, loss = pl.pallas_call(
        _body,
        grid=(2 * NB,),
        in_specs=[
            pl.BlockSpec((T, D), lambda i: (jnp.minimum(i, NB - 1), 0)),
            full((D, HID)), full((HID,)), full((HID, HID // 2)), full((HID // 2,)),
            full((HID // 2, TOTAL)), full((TOTAL,)), full((1, 1)),
            hbm, full((2, D)), full((2, D)), full((2, D)),
            hbm, full((2, MLP_HID)), hbm, full((2, D)),
            hbm, full((2, D)), full((D,)), full((D,)),
        ],
        out_specs=[
            pl.BlockSpec((T, D), lambda i: (jnp.maximum(i - NB, 0), 0)),
            pl.BlockSpec((1, 1), lambda i: (0, 0)),
        ],
        out_shape=[
            jax.ShapeDtypeStruct((S, D), jnp.float32),
            jax.ShapeDtypeStruct((1, 1), jnp.float32),
        ],
        scratch_shapes=[
            pltpu.VMEM((1, TOTAL), jnp.float32),
            pltpu.VMEM((2 * S, D), jnp.bfloat16),
            pltpu.VMEM((S, TOTAL), jnp.float32),
            pltpu.VMEM((D, 2 * D), jnp.bfloat16),
            pltpu.VMEM((D, 2 * MLP_HID), jnp.bfloat16),
            pltpu.VMEM((2 * MLP_HID, D), jnp.bfloat16),
            pltpu.VMEM((2 * D, D), jnp.bfloat16),
            pltpu.VMEM((D, MLP_HID), jnp.float32),
            pltpu.VMEM((D, MLP_HID), jnp.float32),
            pltpu.SemaphoreType.DMA((2,)),
        ],
        compiler_params=pltpu.CompilerParams(
            dimension_semantics=("arbitrary",),
        ),
    )(
        xf, rW1, rb1, rW2, rb2, rW3, rb3, temp.reshape(1, 1),
        W_pre, b_pre, g_pre, be_pre,
        W_m1, b_m1, W_m2, b_m2,
        W_po, b_po, g_po, be_po,
    )
    return out.reshape(1, S, D), loss.reshape(())
